# Initial kernel scaffold; baseline (speedup 1.0000x reference)
import functools

import jax
import jax.numpy as jnp
from jax import lax
from jax.experimental import pallas as pl
from jax.experimental.pallas import tpu as pltpu
from jax.experimental.pallas import tpu_sc as plsc

N = 10000
E = 320000
DF = 128
EMB = 64

NC = 2
NS = 16
NW = NC * NS

G = 80
NBG = E // G
GPT = NBG // NW
RPT = N // NS
RZB = 125

DEGW = 16

BN = 1000
NBLK = N // BN

_mesh = plsc.VectorSubcoreMesh(core_axis_name="c", subcore_axis_name="s")


def _deg_body(dst_hbm, out_hbm, idx_v, ones_v, stg_v, acc_sh):
    c = lax.axis_index("c")
    s = lax.axis_index("s")
    wid = c * NS + s

    pltpu.sync_copy(dst_hbm.at[pl.ds(wid * GPT, GPT)], idx_v)

    def fill_ones(i, carry):
        ones_v[i] = jnp.full((16,), 1.0, jnp.float32)
        return carry

    lax.fori_loop(0, G, fill_ones, 0)

    def fill_zero(i, carry):
        stg_v[i] = jnp.zeros((16,), jnp.float32)
        return carry

    lax.fori_loop(0, RZB, fill_zero, 0)

    for kk in range(RPT // RZB):
        pltpu.sync_copy(stg_v, acc_sh.at[pl.ds(s * RPT + kk * RZB, RZB)])
    plsc.subcore_barrier()

    def scat(j, carry):
        pltpu.sync_copy(ones_v, acc_sh.at[idx_v.at[j]], add=True)
        return carry

    lax.fori_loop(0, GPT, scat, 0)
    plsc.subcore_barrier()

    for kk in range(RPT // RZB):
        r0 = s * RPT + kk * RZB
        pltpu.sync_copy(acc_sh.at[pl.ds(r0, RZB)], stg_v)
        pltpu.sync_copy(stg_v, out_hbm.at[c, pl.ds(r0, RZB)])


_deg_call = functools.partial(
    pl.kernel,
    out_type=jax.ShapeDtypeStruct((NC, N, DEGW), jnp.float32),
    mesh=_mesh,
    scratch_types=[
        pltpu.VMEM((GPT, G), jnp.int32),
        pltpu.VMEM((G, DEGW), jnp.float32),
        pltpu.VMEM((RZB, DEGW), jnp.float32),
        pltpu.VMEM_SHARED((N, DEGW), jnp.float32),
    ],
)(_deg_body)


def _scat_body(y_hbm, src_hbm, dst_hbm, out_hbm, srcv, dstv, rows_v, stg_v,
               acc_sh):
    c = lax.axis_index("c")
    s = lax.axis_index("s")
    wid = c * NS + s

    pltpu.sync_copy(src_hbm.at[pl.ds(wid * GPT, GPT)], srcv)
    pltpu.sync_copy(dst_hbm.at[pl.ds(wid * GPT, GPT)], dstv)

    def fill_zero(i, carry):
        for kk in range(DF // 16):
            stg_v[i, pl.ds(kk * 16, 16)] = jnp.zeros((16,), jnp.float32)
        return carry

    lax.fori_loop(0, RZB, fill_zero, 0)
    for kk in range(RPT // RZB):
        pltpu.sync_copy(stg_v, acc_sh.at[pl.ds(s * RPT + kk * RZB, RZB)])
    plsc.subcore_barrier()

    def step(j, carry):
        pltpu.sync_copy(y_hbm.at[srcv.at[j]], rows_v)
        pltpu.sync_copy(rows_v, acc_sh.at[dstv.at[j]], add=True)
        return carry

    lax.fori_loop(0, GPT, step, 0)
    plsc.subcore_barrier()

    for kk in range(RPT // RZB):
        r0 = s * RPT + kk * RZB
        pltpu.sync_copy(acc_sh.at[pl.ds(r0, RZB)], stg_v)
        pltpu.sync_copy(stg_v, out_hbm.at[c, pl.ds(r0, RZB)])


_scat_call = functools.partial(
    pl.kernel,
    out_type=jax.ShapeDtypeStruct((NC, N, DF), jnp.float32),
    mesh=_mesh,
    scratch_types=[
        pltpu.VMEM((GPT, G), jnp.int32),
        pltpu.VMEM((GPT, G), jnp.int32),
        pltpu.VMEM((G, DF), jnp.float32),
        pltpu.VMEM((RZB, DF), jnp.float32),
        pltpu.VMEM_SHARED((N, DF), jnp.float32),
    ],
)(_scat_body)


def _dinv_of(deg_ref):
    return lax.rsqrt(deg_ref[0, :, 0:1] + deg_ref[1, :, 0:1] + 1.0)


def _lin1_body(deg_ref, x_ref, w_ref, y_ref):
    dinv = _dinv_of(deg_ref)
    xw = jnp.dot(x_ref[...], w_ref[...], preferred_element_type=jnp.float32)
    y_ref[...] = xw * dinv


def _lin2_body(deg_ref, s_ref, y_ref, b_ref, w_ref, o_ref):
    dinv = _dinv_of(deg_ref)
    h = dinv * (s_ref[0] + s_ref[1] + y_ref[...]) + b_ref[...]
    h = jnp.maximum(h, 0.0)
    o_ref[...] = jnp.dot(h, w_ref[...],
                         preferred_element_type=jnp.float32) * dinv


def _fin_body(deg_ref, s_ref, y_ref, b_ref, wl_ref, bl_ref, o_ref):
    dinv = _dinv_of(deg_ref)
    h = dinv * (s_ref[0] + s_ref[1] + y_ref[...]) + b_ref[...]
    h = jnp.maximum(h, 0.0)
    r = lax.dot_general(wl_ref[...], h, (((0,), (1,)), ((), ())),
                        preferred_element_type=jnp.float32)
    o_ref[...] = jax.nn.sigmoid(r + bl_ref[...])


def _deg_spec():
    return pl.BlockSpec((NC, BN, DEGW), lambda i: (0, i, 0))


def _row_spec():
    return pl.BlockSpec((BN, DF), lambda i: (i, 0))


def _part_spec():
    return pl.BlockSpec((NC, BN, DF), lambda i: (0, i, 0))


def _full_spec(shape):
    nd = len(shape)
    return pl.BlockSpec(shape, lambda i: (0,) * nd)


def kernel(node_features, edge_index, W1, b1, Wh, bh, Wl, bl):
    x = node_features.astype(jnp.float32)
    src2d = edge_index[0].reshape(NBG, G)
    dst2d = edge_index[1].reshape(NBG, G)

    degp = _deg_call(dst2d)

    y1 = pl.pallas_call(
        _lin1_body,
        grid=(NBLK,),
        in_specs=[_deg_spec(), _row_spec(), _full_spec((DF, DF))],
        out_specs=_row_spec(),
        out_shape=jax.ShapeDtypeStruct((N, DF), jnp.float32),
    )(degp, x, W1)

    s1 = _scat_call(y1, src2d, dst2d)

    y2 = pl.pallas_call(
        _lin2_body,
        grid=(NBLK,),
        in_specs=[_deg_spec(), _part_spec(), _row_spec(),
                  _full_spec((1, DF)), _full_spec((DF, DF))],
        out_specs=_row_spec(),
        out_shape=jax.ShapeDtypeStruct((N, DF), jnp.float32),
    )(degp, s1, y1, b1.reshape(1, DF), Wh)

    s2 = _scat_call(y2, src2d, dst2d)

    out = pl.pallas_call(
        _fin_body,
        grid=(NBLK,),
        in_specs=[_deg_spec(), _part_spec(), _row_spec(),
                  _full_spec((1, DF)), _full_spec((DF, EMB)),
                  _full_spec((EMB, 1))],
        out_specs=pl.BlockSpec((EMB, BN), lambda i: (0, i)),
        out_shape=jax.ShapeDtypeStruct((EMB, N), jnp.float32),
    )(degp, s2, y2, bh.reshape(1, DF), Wl, bl.reshape(EMB, 1))

    return out


# speedup vs baseline: 18.7415x; 18.7415x over previous
import functools

import jax
import jax.numpy as jnp
from jax import lax
from jax.experimental import pallas as pl
from jax.experimental.pallas import tpu as pltpu
from jax.experimental.pallas import tpu_sc as plsc

N = 10000
E = 320000
DF = 128
EMB = 64

NC = 2
NS = 16
NW = NC * NS

G = 80
EPT = E // NW
GPT = EPT // G
RPT = N // NS

DEGW = 16

BN = 1024
NBLK = (N + BN - 1) // BN

_mesh = plsc.VectorSubcoreMesh(core_axis_name="c", subcore_axis_name="s")


def _deg_body(dst_hbm, out_hbm, idx_v, ones_v, stg_v, acc_sh):
    c = lax.axis_index("c")
    s = lax.axis_index("s")
    wid = c * NS + s

    pltpu.sync_copy(dst_hbm.at[wid], idx_v)

    def fill_ones(i, carry):
        ones_v[i] = jnp.full((16,), 1.0, jnp.float32)
        return carry

    lax.fori_loop(0, G, fill_ones, 0)

    def fill_zero(i, carry):
        stg_v[i] = jnp.zeros((16,), jnp.float32)
        return carry

    lax.fori_loop(0, G, fill_zero, 0)

    for q in range(RPT // G):
        pltpu.sync_copy(stg_v, acc_sh.at[pl.ds(s * RPT + q * G, G)])
    rem = RPT - (RPT // G) * G
    pltpu.sync_copy(stg_v.at[pl.ds(0, rem)],
                    acc_sh.at[pl.ds(s * RPT + RPT - rem, rem)])
    plsc.subcore_barrier()

    def scat(j, carry):
        pltpu.sync_copy(ones_v, acc_sh.at[idx_v.at[j]], add=True)
        return carry

    lax.fori_loop(0, GPT, scat, 0)
    plsc.subcore_barrier()

    pltpu.sync_copy(acc_sh.at[pl.ds(s * RPT, RPT)], out_hbm.at[c, s])


_deg_call = functools.partial(
    pl.kernel,
    out_type=jax.ShapeDtypeStruct((NC, NS, RPT, DEGW), jnp.float32),
    mesh=_mesh,
    scratch_types=[
        pltpu.VMEM((GPT, G), jnp.int32),
        pltpu.VMEM((G, DEGW), jnp.float32),
        pltpu.VMEM((G, DEGW), jnp.float32),
        pltpu.VMEM_SHARED((N, DEGW), jnp.float32),
    ],
)(_deg_body)


def _scat_body(y_hbm, src_hbm, dst_hbm, out_hbm, srcv, dstv, rows_v, acc_sh):
    c = lax.axis_index("c")
    s = lax.axis_index("s")
    wid = c * NS + s

    pltpu.sync_copy(src_hbm.at[wid], srcv)
    pltpu.sync_copy(dst_hbm.at[wid], dstv)

    def fill_zero(i, carry):
        for kk in range(DF // 16):
            rows_v[i, pl.ds(kk * 16, 16)] = jnp.zeros((16,), jnp.float32)
        return carry

    lax.fori_loop(0, G, fill_zero, 0)
    for q in range(RPT // G):
        pltpu.sync_copy(rows_v, acc_sh.at[pl.ds(s * RPT + q * G, G)])
    rem = RPT - (RPT // G) * G
    pltpu.sync_copy(rows_v.at[pl.ds(0, rem)],
                    acc_sh.at[pl.ds(s * RPT + RPT - rem, rem)])
    plsc.subcore_barrier()

    def step(j, carry):
        pltpu.sync_copy(y_hbm.at[srcv.at[j]], rows_v)
        pltpu.sync_copy(rows_v, acc_sh.at[dstv.at[j]], add=True)
        return carry

    lax.fori_loop(0, GPT, step, 0)
    plsc.subcore_barrier()

    pltpu.sync_copy(acc_sh.at[pl.ds(s * RPT, RPT)], out_hbm.at[c, s])


_scat_call = functools.partial(
    pl.kernel,
    out_type=jax.ShapeDtypeStruct((NC, NS, RPT, DF), jnp.float32),
    mesh=_mesh,
    scratch_types=[
        pltpu.VMEM((GPT, G), jnp.int32),
        pltpu.VMEM((GPT, G), jnp.int32),
        pltpu.VMEM((G, DF), jnp.float32),
        pltpu.VMEM_SHARED((N, DF), jnp.float32),
    ],
)(_scat_body)


def _dinv_of(deg_ref):
    return lax.rsqrt(deg_ref[0, :, 0:1] + deg_ref[1, :, 0:1] + 1.0)


def _lin1_body(deg_ref, x_ref, w_ref, y_ref):
    dinv = _dinv_of(deg_ref)
    xw = jnp.dot(x_ref[...], w_ref[...], preferred_element_type=jnp.float32)
    y_ref[...] = xw * dinv


def _lin2_body(deg_ref, s_ref, y_ref, b_ref, w_ref, o_ref):
    dinv = _dinv_of(deg_ref)
    h = dinv * (s_ref[0] + s_ref[1] + y_ref[...]) + b_ref[...]
    h = jnp.maximum(h, 0.0)
    o_ref[...] = jnp.dot(h, w_ref[...],
                         preferred_element_type=jnp.float32) * dinv


def _fin_body(deg_ref, s_ref, y_ref, b_ref, wl_ref, bl_ref, o_ref):
    dinv = _dinv_of(deg_ref)
    h = dinv * (s_ref[0] + s_ref[1] + y_ref[...]) + b_ref[...]
    h = jnp.maximum(h, 0.0)
    r = lax.dot_general(wl_ref[...], h, (((0,), (1,)), ((), ())),
                        preferred_element_type=jnp.float32)
    o_ref[...] = jax.nn.sigmoid(r + bl_ref[...])


def _deg_spec():
    return pl.BlockSpec((NC, BN, DEGW), lambda i: (0, i, 0))


def _row_spec():
    return pl.BlockSpec((BN, DF), lambda i: (i, 0))


def _part_spec():
    return pl.BlockSpec((NC, BN, DF), lambda i: (0, i, 0))


def _full_spec(shape):
    nd = len(shape)
    return pl.BlockSpec(shape, lambda i: (0,) * nd)


def kernel(node_features, edge_index, W1, b1, Wh, bh, Wl, bl):
    x = node_features.astype(jnp.float32)
    src3d = edge_index[0].reshape(NW, GPT, G)
    dst3d = edge_index[1].reshape(NW, GPT, G)

    degp = _deg_call(dst3d).reshape(NC, N, DEGW)

    y1 = pl.pallas_call(
        _lin1_body,
        grid=(NBLK,),
        in_specs=[_deg_spec(), _row_spec(), _full_spec((DF, DF))],
        out_specs=_row_spec(),
        out_shape=jax.ShapeDtypeStruct((N, DF), jnp.float32),
    )(degp, x, W1)

    s1 = _scat_call(y1, src3d, dst3d).reshape(NC, N, DF)

    y2 = pl.pallas_call(
        _lin2_body,
        grid=(NBLK,),
        in_specs=[_deg_spec(), _part_spec(), _row_spec(),
                  _full_spec((1, DF)), _full_spec((DF, DF))],
        out_specs=_row_spec(),
        out_shape=jax.ShapeDtypeStruct((N, DF), jnp.float32),
    )(degp, s1, y1, b1.reshape(1, DF), Wh)

    s2 = _scat_call(y2, src3d, dst3d).reshape(NC, N, DF)

    out = pl.pallas_call(
        _fin_body,
        grid=(NBLK,),
        in_specs=[_deg_spec(), _part_spec(), _row_spec(),
                  _full_spec((1, DF)), _full_spec((DF, EMB)),
                  _full_spec((EMB, 1))],
        out_specs=pl.BlockSpec((EMB, BN), lambda i: (0, i)),
        out_shape=jax.ShapeDtypeStruct((EMB, N), jnp.float32),
    )(degp, s2, y2, bh.reshape(1, DF), Wl, bl.reshape(EMB, 1))

    return out


# speedup vs baseline: 29.1396x; 1.5548x over previous
import functools

import jax
import jax.numpy as jnp
from jax import lax
from jax.experimental import pallas as pl
from jax.experimental.pallas import tpu as pltpu
from jax.experimental.pallas import tpu_sc as plsc

N = 10000
E = 320000
DF = 128
EMB = 64

NC = 2
NS = 16
NW = NC * NS

G = 100
EPT = E // NW
GPT = EPT // G
NH = 2
HG = GPT // NH
RPT = N // NS

DEGW = 16

BN = 1024
NBLK = (N + BN - 1) // BN

_mesh = plsc.VectorSubcoreMesh(core_axis_name="c", subcore_axis_name="s")


def _deg_body(dst_hbm, out_hbm, idx_v, ones_v, stg_v, acc_sh, sem):
    c = lax.axis_index("c")
    s = lax.axis_index("s")
    wid = c * NS + s

    pltpu.sync_copy(dst_hbm.at[wid], idx_v)

    def fill_ones(i, carry):
        ones_v[i] = jnp.full((16,), 1.0, jnp.float32)
        return carry

    lax.fori_loop(0, G, fill_ones, 0)

    def fill_zero(i, carry):
        stg_v[i] = jnp.zeros((16,), jnp.float32)
        return carry

    lax.fori_loop(0, G, fill_zero, 0)

    for q in range(RPT // G):
        pltpu.sync_copy(stg_v, acc_sh.at[pl.ds(s * RPT + q * G, G)])
    rem = RPT - (RPT // G) * G
    pltpu.sync_copy(stg_v.at[pl.ds(0, rem)],
                    acc_sh.at[pl.ds(s * RPT + RPT - rem, rem)])
    plsc.subcore_barrier()

    def scat(j, carry):
        pltpu.sync_copy(ones_v, acc_sh.at[idx_v.at[j]], add=True)
        return carry

    lax.fori_loop(0, GPT, scat, 0)
    plsc.subcore_barrier()

    pltpu.sync_copy(acc_sh.at[pl.ds(s * RPT, RPT)], out_hbm.at[c, s])


_deg_call = functools.partial(
    pl.kernel,
    out_type=jax.ShapeDtypeStruct((NC, NS, RPT, DEGW), jnp.float32),
    mesh=_mesh,
    scratch_types=[
        pltpu.VMEM((GPT, G), jnp.int32),
        pltpu.VMEM((G, DEGW), jnp.float32),
        pltpu.VMEM((G, DEGW), jnp.float32),
        pltpu.VMEM_SHARED((N, DEGW), jnp.float32),
        pltpu.SemaphoreType.DMA,
    ],
)(_deg_body)


def _scat_body(y_hbm, src_hbm, dst_hbm, out_hbm, srcv, dstv, rows0_v, rows1_v,
               acc_sh, sem0, sem1):
    c = lax.axis_index("c")
    s = lax.axis_index("s")
    wid = c * NS + s

    pltpu.sync_copy(src_hbm.at[wid, 0], srcv)
    pltpu.sync_copy(dst_hbm.at[wid, 0], dstv)

    def fill_zero(i, carry):
        for kk in range(DF // 16):
            rows0_v[i, pl.ds(kk * 16, 16)] = jnp.zeros((16,), jnp.float32)
        return carry

    lax.fori_loop(0, G, fill_zero, 0)
    for q in range(RPT // G):
        pltpu.sync_copy(rows0_v, acc_sh.at[pl.ds(s * RPT + q * G, G)])
    rem = RPT - (RPT // G) * G
    pltpu.sync_copy(rows0_v.at[pl.ds(0, rem)],
                    acc_sh.at[pl.ds(s * RPT + RPT - rem, rem)])
    def step(t, carry):
        j0 = 2 * t
        j1 = j0 + 1
        pltpu.async_copy(y_hbm.at[srcv.at[j1]], rows1_v, sem1)
        pltpu.make_async_copy(y_hbm.at[srcv.at[j0]], rows0_v, sem0).wait()
        pltpu.sync_copy(rows0_v, acc_sh.at[dstv.at[j0]], add=True)

        @pl.when(j1 + 1 < HG)
        def _():
            pltpu.async_copy(y_hbm.at[srcv.at[j1 + 1]], rows0_v, sem0)

        pltpu.make_async_copy(y_hbm.at[srcv.at[j1]], rows1_v, sem1).wait()
        pltpu.sync_copy(rows1_v, acc_sh.at[dstv.at[j1]], add=True)
        return carry

    pltpu.async_copy(y_hbm.at[srcv.at[0]], rows0_v, sem0)
    plsc.subcore_barrier()
    lax.fori_loop(0, HG // 2, step, 0)

    pltpu.sync_copy(src_hbm.at[wid, 1], srcv)
    pltpu.sync_copy(dst_hbm.at[wid, 1], dstv)
    pltpu.async_copy(y_hbm.at[srcv.at[0]], rows0_v, sem0)
    lax.fori_loop(0, HG // 2, step, 0)
    plsc.subcore_barrier()

    pltpu.sync_copy(acc_sh.at[pl.ds(s * RPT, RPT)], out_hbm.at[c, s])


_scat_call = functools.partial(
    pl.kernel,
    out_type=jax.ShapeDtypeStruct((NC, NS, RPT, DF), jnp.float32),
    mesh=_mesh,
    scratch_types=[
        pltpu.VMEM((HG, G), jnp.int32),
        pltpu.VMEM((HG, G), jnp.int32),
        pltpu.VMEM((G, DF), jnp.float32),
        pltpu.VMEM((G, DF), jnp.float32),
        pltpu.VMEM_SHARED((N, DF), jnp.float32),
        pltpu.SemaphoreType.DMA,
        pltpu.SemaphoreType.DMA,
    ],
)(_scat_body)


def _dinv_of(deg_ref):
    return lax.rsqrt(deg_ref[0, :, 0:1] + deg_ref[1, :, 0:1] + 1.0)


def _lin1_body(deg_ref, x_ref, w_ref, y_ref):
    dinv = _dinv_of(deg_ref)
    xw = jnp.dot(x_ref[...], w_ref[...], preferred_element_type=jnp.float32)
    y_ref[...] = xw * dinv


def _lin2_body(deg_ref, s_ref, y_ref, b_ref, w_ref, o_ref):
    dinv = _dinv_of(deg_ref)
    h = dinv * (s_ref[0] + s_ref[1] + y_ref[...]) + b_ref[...]
    h = jnp.maximum(h, 0.0)
    o_ref[...] = jnp.dot(h, w_ref[...],
                         preferred_element_type=jnp.float32) * dinv


def _fin_body(deg_ref, s_ref, y_ref, b_ref, wl_ref, bl_ref, o_ref):
    dinv = _dinv_of(deg_ref)
    h = dinv * (s_ref[0] + s_ref[1] + y_ref[...]) + b_ref[...]
    h = jnp.maximum(h, 0.0)
    r = lax.dot_general(wl_ref[...], h, (((0,), (1,)), ((), ())),
                        preferred_element_type=jnp.float32)
    o_ref[...] = jax.nn.sigmoid(r + bl_ref[...])


def _deg_spec():
    return pl.BlockSpec((NC, BN, DEGW), lambda i: (0, i, 0))


def _row_spec():
    return pl.BlockSpec((BN, DF), lambda i: (i, 0))


def _part_spec():
    return pl.BlockSpec((NC, BN, DF), lambda i: (0, i, 0))


def _full_spec(shape):
    nd = len(shape)
    return pl.BlockSpec(shape, lambda i: (0,) * nd)


def kernel(node_features, edge_index, W1, b1, Wh, bh, Wl, bl):
    x = node_features.astype(jnp.float32)
    src4d = edge_index[0].reshape(NW, NH, HG, G)
    dst4d = edge_index[1].reshape(NW, NH, HG, G)

    degp = _deg_call(edge_index[1].reshape(NW, GPT, G)).reshape(NC, N, DEGW)

    y1 = pl.pallas_call(
        _lin1_body,
        grid=(NBLK,),
        in_specs=[_deg_spec(), _row_spec(), _full_spec((DF, DF))],
        out_specs=_row_spec(),
        out_shape=jax.ShapeDtypeStruct((N, DF), jnp.float32),
    )(degp, x, W1)

    s1 = _scat_call(y1, src4d, dst4d).reshape(NC, N, DF)

    y2 = pl.pallas_call(
        _lin2_body,
        grid=(NBLK,),
        in_specs=[_deg_spec(), _part_spec(), _row_spec(),
                  _full_spec((1, DF)), _full_spec((DF, DF))],
        out_specs=_row_spec(),
        out_shape=jax.ShapeDtypeStruct((N, DF), jnp.float32),
    )(degp, s1, y1, b1.reshape(1, DF), Wh)

    s2 = _scat_call(y2, src4d, dst4d).reshape(NC, N, DF)

    out = pl.pallas_call(
        _fin_body,
        grid=(NBLK,),
        in_specs=[_deg_spec(), _part_spec(), _row_spec(),
                  _full_spec((1, DF)), _full_spec((DF, EMB)),
                  _full_spec((EMB, 1))],
        out_specs=pl.BlockSpec((EMB, BN), lambda i: (0, i)),
        out_shape=jax.ShapeDtypeStruct((EMB, N), jnp.float32),
    )(degp, s2, y2, bh.reshape(1, DF), Wl, bl.reshape(EMB, 1))

    return out


# speedup vs baseline: 29.4435x; 1.0104x over previous
import functools

import jax
import jax.numpy as jnp
from jax import lax
from jax.experimental import pallas as pl
from jax.experimental.pallas import tpu as pltpu
from jax.experimental.pallas import tpu_sc as plsc

N = 10000
E = 320000
DF = 128
EMB = 64

NC = 2
NS = 16
NW = NC * NS

G = 100
EPT = E // NW
GPT = EPT // G
NH = 2
HG = GPT // NH
RPT = N // NS

DEGW = 16

BN = 1024
NBLK = (N + BN - 1) // BN

_mesh = plsc.VectorSubcoreMesh(core_axis_name="c", subcore_axis_name="s")


def _deg_body(dst_hbm, out_hbm, idx_v, ones_v, stg_v, acc_sh, sem, sem2):
    c = lax.axis_index("c")
    s = lax.axis_index("s")
    wid = c * NS + s

    pltpu.sync_copy(dst_hbm.at[wid], idx_v)

    def fill_ones(i, carry):
        ones_v[i] = jnp.full((16,), 1.0, jnp.float32)
        return carry

    lax.fori_loop(0, G, fill_ones, 0)

    def fill_zero(i, carry):
        stg_v[i] = jnp.zeros((16,), jnp.float32)
        return carry

    lax.fori_loop(0, G, fill_zero, 0)

    for q in range(RPT // G):
        pltpu.sync_copy(stg_v, acc_sh.at[pl.ds(s * RPT + q * G, G)])
    rem = RPT - (RPT // G) * G
    pltpu.sync_copy(stg_v.at[pl.ds(0, rem)],
                    acc_sh.at[pl.ds(s * RPT + RPT - rem, rem)])
    plsc.subcore_barrier()

    pltpu.async_copy(ones_v, acc_sh.at[idx_v.at[0]], sem, add=True)
    pltpu.async_copy(ones_v, acc_sh.at[idx_v.at[1]], sem2, add=True)

    def scat(t, carry):
        j0 = 2 * t
        j1 = j0 + 1
        pltpu.make_async_copy(ones_v, acc_sh.at[idx_v.at[j0]], sem).wait()

        @pl.when(j0 + 2 < GPT)
        def _():
            pltpu.async_copy(ones_v, acc_sh.at[idx_v.at[j0 + 2]], sem,
                             add=True)

        pltpu.make_async_copy(ones_v, acc_sh.at[idx_v.at[j1]], sem2).wait()

        @pl.when(j1 + 2 < GPT)
        def _():
            pltpu.async_copy(ones_v, acc_sh.at[idx_v.at[j1 + 2]], sem2,
                             add=True)

        return carry

    lax.fori_loop(0, GPT // 2, scat, 0)
    plsc.subcore_barrier()

    pltpu.sync_copy(acc_sh.at[pl.ds(s * RPT, RPT)], out_hbm.at[c, s])


_deg_call = functools.partial(
    pl.kernel,
    out_type=jax.ShapeDtypeStruct((NC, NS, RPT, DEGW), jnp.float32),
    mesh=_mesh,
    scratch_types=[
        pltpu.VMEM((GPT, G), jnp.int32),
        pltpu.VMEM((G, DEGW), jnp.float32),
        pltpu.VMEM((G, DEGW), jnp.float32),
        pltpu.VMEM_SHARED((N, DEGW), jnp.float32),
        pltpu.SemaphoreType.DMA,
        pltpu.SemaphoreType.DMA,
    ],
)(_deg_body)


def _scat_body(y_hbm, src_hbm, dst_hbm, out_hbm, srcv, dstv, rows0_v, rows1_v,
               acc_sh, sem0, sem1):
    c = lax.axis_index("c")
    s = lax.axis_index("s")
    wid = c * NS + s

    pltpu.sync_copy(src_hbm.at[wid, 0], srcv)
    pltpu.sync_copy(dst_hbm.at[wid, 0], dstv)

    def fill_zero(i, carry):
        for kk in range(DF // 16):
            rows0_v[i, pl.ds(kk * 16, 16)] = jnp.zeros((16,), jnp.float32)
        return carry

    lax.fori_loop(0, G, fill_zero, 0)
    for q in range(RPT // G):
        pltpu.sync_copy(rows0_v, acc_sh.at[pl.ds(s * RPT + q * G, G)])
    rem = RPT - (RPT // G) * G
    pltpu.sync_copy(rows0_v.at[pl.ds(0, rem)],
                    acc_sh.at[pl.ds(s * RPT + RPT - rem, rem)])
    def step(t, carry):
        j0 = 2 * t
        j1 = j0 + 1
        pltpu.async_copy(y_hbm.at[srcv.at[j1]], rows1_v, sem1)
        pltpu.make_async_copy(y_hbm.at[srcv.at[j0]], rows0_v, sem0).wait()
        pltpu.sync_copy(rows0_v, acc_sh.at[dstv.at[j0]], add=True)

        @pl.when(j1 + 1 < HG)
        def _():
            pltpu.async_copy(y_hbm.at[srcv.at[j1 + 1]], rows0_v, sem0)

        pltpu.make_async_copy(y_hbm.at[srcv.at[j1]], rows1_v, sem1).wait()
        pltpu.sync_copy(rows1_v, acc_sh.at[dstv.at[j1]], add=True)
        return carry

    pltpu.async_copy(y_hbm.at[srcv.at[0]], rows0_v, sem0)
    plsc.subcore_barrier()
    lax.fori_loop(0, HG // 2, step, 0)

    pltpu.sync_copy(src_hbm.at[wid, 1], srcv)
    pltpu.sync_copy(dst_hbm.at[wid, 1], dstv)
    pltpu.async_copy(y_hbm.at[srcv.at[0]], rows0_v, sem0)
    lax.fori_loop(0, HG // 2, step, 0)
    plsc.subcore_barrier()

    pltpu.sync_copy(acc_sh.at[pl.ds(s * RPT, RPT)], out_hbm.at[c, s])


_scat_call = functools.partial(
    pl.kernel,
    out_type=jax.ShapeDtypeStruct((NC, NS, RPT, DF), jnp.float32),
    mesh=_mesh,
    scratch_types=[
        pltpu.VMEM((HG, G), jnp.int32),
        pltpu.VMEM((HG, G), jnp.int32),
        pltpu.VMEM((G, DF), jnp.float32),
        pltpu.VMEM((G, DF), jnp.float32),
        pltpu.VMEM_SHARED((N, DF), jnp.float32),
        pltpu.SemaphoreType.DMA,
        pltpu.SemaphoreType.DMA,
    ],
)(_scat_body)


def _dinv_of(deg_ref):
    return lax.rsqrt(deg_ref[0, :, 0:1] + deg_ref[1, :, 0:1] + 1.0)


def _lin1_body(deg_ref, x_ref, w_ref, y_ref):
    dinv = _dinv_of(deg_ref)
    xw = jnp.dot(x_ref[...], w_ref[...], preferred_element_type=jnp.float32)
    y_ref[...] = xw * dinv


def _lin2_body(deg_ref, s_ref, y_ref, b_ref, w_ref, o_ref):
    dinv = _dinv_of(deg_ref)
    h = dinv * (s_ref[0] + s_ref[1] + y_ref[...]) + b_ref[...]
    h = jnp.maximum(h, 0.0)
    o_ref[...] = jnp.dot(h, w_ref[...],
                         preferred_element_type=jnp.float32) * dinv


def _fin_body(deg_ref, s_ref, y_ref, b_ref, wl_ref, bl_ref, o_ref):
    dinv = _dinv_of(deg_ref)
    h = dinv * (s_ref[0] + s_ref[1] + y_ref[...]) + b_ref[...]
    h = jnp.maximum(h, 0.0)
    r = lax.dot_general(wl_ref[...], h, (((0,), (1,)), ((), ())),
                        preferred_element_type=jnp.float32)
    o_ref[...] = jax.nn.sigmoid(r + bl_ref[...])


def _deg_spec():
    return pl.BlockSpec((NC, BN, DEGW), lambda i: (0, i, 0))


def _row_spec():
    return pl.BlockSpec((BN, DF), lambda i: (i, 0))


def _part_spec():
    return pl.BlockSpec((NC, BN, DF), lambda i: (0, i, 0))


def _full_spec(shape):
    nd = len(shape)
    return pl.BlockSpec(shape, lambda i: (0,) * nd)


def kernel(node_features, edge_index, W1, b1, Wh, bh, Wl, bl):
    x = node_features.astype(jnp.float32)
    src4d = edge_index[0].reshape(NW, NH, HG, G)
    dst4d = edge_index[1].reshape(NW, NH, HG, G)

    degp = _deg_call(edge_index[1].reshape(NW, GPT, G)).reshape(NC, N, DEGW)

    y1 = pl.pallas_call(
        _lin1_body,
        grid=(NBLK,),
        in_specs=[_deg_spec(), _row_spec(), _full_spec((DF, DF))],
        out_specs=_row_spec(),
        out_shape=jax.ShapeDtypeStruct((N, DF), jnp.float32),
    )(degp, x, W1)

    s1 = _scat_call(y1, src4d, dst4d).reshape(NC, N, DF)

    y2 = pl.pallas_call(
        _lin2_body,
        grid=(NBLK,),
        in_specs=[_deg_spec(), _part_spec(), _row_spec(),
                  _full_spec((1, DF)), _full_spec((DF, DF))],
        out_specs=_row_spec(),
        out_shape=jax.ShapeDtypeStruct((N, DF), jnp.float32),
    )(degp, s1, y1, b1.reshape(1, DF), Wh)

    s2 = _scat_call(y2, src4d, dst4d).reshape(NC, N, DF)

    out = pl.pallas_call(
        _fin_body,
        grid=(NBLK,),
        in_specs=[_deg_spec(), _part_spec(), _row_spec(),
                  _full_spec((1, DF)), _full_spec((DF, EMB)),
                  _full_spec((EMB, 1))],
        out_specs=pl.BlockSpec((EMB, BN), lambda i: (0, i)),
        out_shape=jax.ShapeDtypeStruct((EMB, N), jnp.float32),
    )(degp, s2, y2, bh.reshape(1, DF), Wl, bl.reshape(EMB, 1))

    return out
